# Initial kernel scaffold; baseline (speedup 1.0000x reference)
import jax
import jax.numpy as jnp
from jax.experimental import pallas as pl


def kernel(x, edge_index0, edge_type0, edge_index1, edge_type1, W0, Wself0, b0, bn0_g, bn0_b, W1, Wself1, b1, bn1_g, bn1_b, mlp_W1, mlp_b1, mlp_bn_g, mlp_bn_b, mlp_W2, mlp_b2):
    raise NotImplementedError("write your pallas kernel here")


# speedup vs baseline: 11.4902x; 11.4902x over previous
import functools

import jax
import jax.numpy as jnp
from jax import lax
from jax.experimental import pallas as pl
from jax.experimental.pallas import tpu as pltpu
from jax.experimental.pallas import tpu_sc as plsc

NC = 2
NS = 16
C = 128
NBLK = 64


def _trans(x, W):
    R, D, H = W.shape
    N = x.shape[0]

    def body(x_ref, w_ref, o_ref):
        o_ref[0] = jnp.dot(x_ref[...], w_ref[0],
                           preferred_element_type=jnp.float32)

    return pl.pallas_call(
        body,
        grid=(R,),
        in_specs=[
            pl.BlockSpec((N, D), lambda r: (0, 0)),
            pl.BlockSpec((1, D, H), lambda r: (r, 0, 0)),
        ],
        out_specs=pl.BlockSpec((1, N, H), lambda r: (r, 0, 0)),
        out_shape=jax.ShapeDtypeStruct((R, N, H), jnp.float32),
    )(x, W)


def _flatidx(typ2d, src2d, n_nodes):
    def body(t_ref, s_ref, o_ref):
        o_ref[...] = t_ref[...] * n_nodes + s_ref[...]

    return pl.pallas_call(
        body,
        out_shape=jax.ShapeDtypeStruct(typ2d.shape, jnp.int32),
    )(typ2d, src2d)


def _sc_aggregate(trans_flat, flat3d, dst3d, zacc, zdeg, n_pad):
    RN, H = trans_flat.shape
    _, K2, Cc = flat3d.shape
    K = K2 // 2
    RPW = n_pad // NS
    HG = H // 16

    mesh = plsc.VectorSubcoreMesh(core_axis_name="c", subcore_axis_name="s")

    @functools.partial(
        pl.kernel,
        mesh=mesh,
        out_type=jax.ShapeDtypeStruct((NC, n_pad, H), jnp.float32),
        scratch_types=[
            pltpu.VMEM((K, Cc), jnp.int32),
            pltpu.VMEM((K, Cc), jnp.int32),
            pltpu.VMEM((Cc, H), jnp.float32),
            pltpu.VMEM((Cc,), jnp.float32),
            pltpu.VMEM((NBLK, H), jnp.float32),
            pltpu.VMEM((RPW,), jnp.float32),
            pltpu.VMEM_SHARED((n_pad, H), jnp.float32),
            pltpu.VMEM_SHARED((n_pad,), jnp.float32),
        ],
    )
    def k(trans_hbm, flat_hbm, dst_hbm, zacc_hbm, zdeg_hbm, acc_out,
          flat_v, dst_v, rows_v, ones_v, nblk_v, deg_v, acc_sh, deg_sh):
        c = lax.axis_index("c")
        s = lax.axis_index("s")
        half = pl.multiple_of(c * K, 8)
        other = pl.multiple_of((1 - c) * K, 8)
        rbase = pl.multiple_of(s * RPW, 8)

        pltpu.sync_copy(zacc_hbm.at[pl.ds(rbase, RPW)],
                        acc_sh.at[pl.ds(rbase, RPW)])
        pltpu.sync_copy(zdeg_hbm.at[pl.ds(rbase, RPW)],
                        deg_sh.at[pl.ds(rbase, RPW)])

        pltpu.sync_copy(flat_hbm.at[s, pl.ds(half, K)], flat_v)
        pltpu.sync_copy(dst_hbm.at[s, pl.ds(half, K)], dst_v)

        @pl.loop(0, Cc, step=16)
        def _(i):
            ones_v[pl.ds(i, 16)] = jnp.ones((16,), jnp.float32)

        plsc.subcore_barrier()

        @pl.loop(0, K)
        def _(j):
            pltpu.sync_copy(trans_hbm.at[flat_v.at[j]], rows_v)
            pltpu.sync_copy(rows_v, acc_sh.at[dst_v.at[j]], add=True)
            pltpu.sync_copy(ones_v, deg_sh.at[dst_v.at[j]], add=True)

        pltpu.sync_copy(dst_hbm.at[s, pl.ds(other, K)], dst_v)

        @pl.loop(0, K)
        def _(j):
            pltpu.sync_copy(ones_v, deg_sh.at[dst_v.at[j]], add=True)

        plsc.subcore_barrier()

        pltpu.sync_copy(deg_sh.at[pl.ds(rbase, RPW)], deg_v)

        @pl.loop(0, RPW // NBLK)
        def _(b):
            blk = pl.multiple_of(rbase + b * NBLK, 8)
            pltpu.sync_copy(acc_sh.at[pl.ds(blk, NBLK)], nblk_v)

            @pl.loop(0, NBLK // 16)
            def _(g):
                d = deg_v[pl.ds(b * NBLK + g * 16, 16)]
                rec = 1.0 / jnp.maximum(d, 1.0)
                dnums = lax.GatherDimensionNumbers(
                    offset_dims=(), collapsed_slice_dims=(0,),
                    start_index_map=(0,))
                for l in range(16):
                    r = g * 16 + l
                    rl = lax.gather(
                        rec, jnp.full((16, 1), l, jnp.int32), dnums,
                        slice_sizes=(1,),
                        mode=lax.GatherScatterMode.PROMISE_IN_BOUNDS)
                    for hgrp in range(HG):
                        nblk_v[r, pl.ds(hgrp * 16, 16)] = (
                            nblk_v[r, pl.ds(hgrp * 16, 16)] * rl)

            pltpu.sync_copy(nblk_v, acc_out.at[c, pl.ds(blk, NBLK)])

    return k(trans_flat, flat3d, dst3d, zacc, zdeg)


def _post(acc, x, Wself, b, g, bb):
    N, H = x.shape[0], Wself.shape[1]

    def body(acc_ref, x_ref, w_ref, b_ref, g_ref, bb_ref, o_ref):
        h = (acc_ref[0, :N] + acc_ref[1, :N]
             + jnp.dot(x_ref[...], w_ref[...],
                       preferred_element_type=jnp.float32)
             + b_ref[...])
        mu = jnp.mean(h, axis=0, keepdims=True)
        var = jnp.mean((h - mu) ** 2, axis=0, keepdims=True)
        h = (h - mu) * lax.rsqrt(var + 1e-5) * g_ref[...] + bb_ref[...]
        o_ref[...] = jnp.where(h > 0, h, jnp.exp(jnp.minimum(h, 0.0)) - 1.0)

    return pl.pallas_call(
        body,
        out_shape=jax.ShapeDtypeStruct((N, H), jnp.float32),
    )(acc, x, Wself, b, g, bb)


def _final(acc, x, Wself, b, g, bb, mW1, mb1, mg, mbb, mW2, mb2):
    N = x.shape[0]
    D_OUT = mW2.shape[1]

    def body(acc_ref, x_ref, w_ref, b_ref, g_ref, bb_ref,
             mW1_ref, mb1_ref, mg_ref, mbb_ref, mW2_ref, mb2_ref, o_ref):
        h = (acc_ref[0, :N] + acc_ref[1, :N]
             + jnp.dot(x_ref[...], w_ref[...],
                       preferred_element_type=jnp.float32)
             + b_ref[...])
        mu = jnp.mean(h, axis=0, keepdims=True)
        var = jnp.mean((h - mu) ** 2, axis=0, keepdims=True)
        h = (h - mu) * lax.rsqrt(var + 1e-5) * g_ref[...] + bb_ref[...]
        h = jnp.where(h > 0, h, jnp.exp(jnp.minimum(h, 0.0)) - 1.0)
        m = jnp.dot(h, mW1_ref[...],
                    preferred_element_type=jnp.float32) + mb1_ref[...]
        mu2 = jnp.mean(m, axis=0, keepdims=True)
        var2 = jnp.mean((m - mu2) ** 2, axis=0, keepdims=True)
        m = (m - mu2) * lax.rsqrt(var2 + 1e-5) * mg_ref[...] + mbb_ref[...]
        m = jnp.maximum(m, 0.0)
        o_ref[...] = jnp.dot(m, mW2_ref[...],
                             preferred_element_type=jnp.float32) + mb2_ref[...]

    return pl.pallas_call(
        body,
        out_shape=jax.ShapeDtypeStruct((N, D_OUT), jnp.float32),
    )(acc, x, Wself, b, g, bb, mW1, mb1, mg, mbb, mW2, mb2)


def kernel(x, edge_index0, edge_type0, edge_index1, edge_type1,
           W0, Wself0, b0, bn0_g, bn0_b,
           W1, Wself1, b1, bn1_g, bn1_b,
           mlp_W1, mlp_b1, mlp_bn_g, mlp_bn_b, mlp_W2, mlp_b2):
    N = x.shape[0]
    E = edge_type0.shape[0]
    R, _, H = W0.shape
    NPAD = ((N + 16 * NS - 1) // (16 * NS)) * (16 * NS)
    K = (E + NS * 2 * C - 1) // (NS * 2 * C)
    K = ((K + 7) // 8) * 8
    EPAD = NS * 2 * K * C

    zacc = jnp.zeros((NPAD, H), jnp.float32)
    zdeg = jnp.zeros((NPAD,), jnp.float32)

    def edges3d(edge_index, edge_type):
        pad = EPAD - E
        src = jnp.concatenate([edge_index[0], jnp.zeros((pad,), jnp.int32)])
        typ = jnp.concatenate([edge_type, jnp.zeros((pad,), jnp.int32)])
        dst = jnp.concatenate(
            [edge_index[1], jnp.full((pad,), NPAD - 1, jnp.int32)])
        flat = _flatidx(typ.reshape(EPAD // C, C), src.reshape(EPAD // C, C),
                        N)
        return flat.reshape(NS, 2 * K, C), dst.reshape(NS, 2 * K, C)

    flat0, dst0 = edges3d(edge_index0, edge_type0)
    flat1, dst1 = edges3d(edge_index1, edge_type1)

    r1h = lambda v: v.reshape(1, -1)

    trans0 = _trans(x, W0).reshape(R * N, H)
    acc0 = _sc_aggregate(trans0, flat0, dst0, zacc, zdeg, NPAD)
    h = _post(acc0, x, Wself0, r1h(b0), r1h(bn0_g), r1h(bn0_b))

    trans1 = _trans(h, W1).reshape(R * N, H)
    acc1 = _sc_aggregate(trans1, flat1, dst1, zacc, zdeg, NPAD)
    out = _final(acc1, h, Wself1, r1h(b1), r1h(bn1_g), r1h(bn1_b),
                 mlp_W1, r1h(mlp_b1), r1h(mlp_bn_g), r1h(mlp_bn_b),
                 mlp_W2, r1h(mlp_b2))
    return out


# speedup vs baseline: 24.6557x; 2.1458x over previous
import functools

import jax
import jax.numpy as jnp
from jax import lax
from jax.experimental import pallas as pl
from jax.experimental.pallas import tpu as pltpu
from jax.experimental.pallas import tpu_sc as plsc

NC = 2
NS = 16
C = 128
NBLK = 64


def _trans(x, W):
    R, D, H = W.shape
    N = x.shape[0]

    def body(x_ref, w_ref, o_ref):
        o_ref[0] = jnp.dot(x_ref[...], w_ref[0],
                           preferred_element_type=jnp.float32)

    return pl.pallas_call(
        body,
        grid=(R,),
        in_specs=[
            pl.BlockSpec((N, D), lambda r: (0, 0)),
            pl.BlockSpec((1, D, H), lambda r: (r, 0, 0)),
        ],
        out_specs=pl.BlockSpec((1, N, H), lambda r: (r, 0, 0)),
        out_shape=jax.ShapeDtypeStruct((R, N, H), jnp.float32),
    )(x, W)


def _flatidx(typ2d, src2d, n_nodes):
    def body(t_ref, s_ref, o_ref):
        o_ref[...] = t_ref[...] * n_nodes + s_ref[...]

    return pl.pallas_call(
        body,
        out_shape=jax.ShapeDtypeStruct(typ2d.shape, jnp.int32),
    )(typ2d, src2d)


def _sc_aggregate(trans_flat, flat3d, dst3d, zacc, zdeg, n_pad):
    RN, H = trans_flat.shape
    _, K2, Cc = flat3d.shape
    K = K2 // 2
    RPW = n_pad // NS
    HG = H // 16

    mesh = plsc.VectorSubcoreMesh(core_axis_name="c", subcore_axis_name="s")

    @functools.partial(
        pl.kernel,
        mesh=mesh,
        out_type=jax.ShapeDtypeStruct((NC, n_pad, H), jnp.float32),
        scratch_types=[
            pltpu.VMEM((K, Cc), jnp.int32),
            pltpu.VMEM((K, Cc), jnp.int32),
            pltpu.VMEM((Cc, H), jnp.float32),
            pltpu.VMEM((Cc,), jnp.float32),
            pltpu.VMEM((NBLK, H), jnp.float32),
            pltpu.VMEM((RPW,), jnp.float32),
            pltpu.VMEM_SHARED((n_pad, H), jnp.float32),
            pltpu.VMEM_SHARED((n_pad,), jnp.float32),
        ],
    )
    def k(trans_hbm, flat_hbm, dst_hbm, zacc_hbm, zdeg_hbm, acc_out,
          flat_v, dst_v, rows_v, ones_v, nblk_v, deg_v, acc_sh, deg_sh):
        c = lax.axis_index("c")
        s = lax.axis_index("s")
        half = pl.multiple_of(c * K, 8)
        other = pl.multiple_of((1 - c) * K, 8)
        rbase = pl.multiple_of(s * RPW, 8)

        pltpu.sync_copy(zacc_hbm.at[pl.ds(rbase, RPW)],
                        acc_sh.at[pl.ds(rbase, RPW)])
        pltpu.sync_copy(zdeg_hbm.at[pl.ds(rbase, RPW)],
                        deg_sh.at[pl.ds(rbase, RPW)])

        pltpu.sync_copy(flat_hbm.at[s, pl.ds(half, K)], flat_v)
        pltpu.sync_copy(dst_hbm.at[s, pl.ds(half, K)], dst_v)

        @pl.loop(0, Cc, step=16)
        def _(i):
            ones_v[pl.ds(i, 16)] = jnp.ones((16,), jnp.float32)

        plsc.subcore_barrier()

        @pl.loop(0, K)
        def _(j):
            pltpu.sync_copy(trans_hbm.at[flat_v.at[j]], rows_v)
            pltpu.sync_copy(rows_v, acc_sh.at[dst_v.at[j]], add=True)
            pltpu.sync_copy(ones_v, deg_sh.at[dst_v.at[j]], add=True)

        pltpu.sync_copy(dst_hbm.at[s, pl.ds(other, K)], dst_v)

        @pl.loop(0, K)
        def _(j):
            pltpu.sync_copy(ones_v, deg_sh.at[dst_v.at[j]], add=True)

        plsc.subcore_barrier()

        pltpu.sync_copy(deg_sh.at[pl.ds(rbase, RPW)], deg_v)

        @pl.loop(0, RPW // NBLK)
        def _(b):
            blk = pl.multiple_of(rbase + b * NBLK, 8)
            pltpu.sync_copy(acc_sh.at[pl.ds(blk, NBLK)], nblk_v)

            @pl.loop(0, NBLK // 16)
            def _(g):
                d = deg_v[pl.ds(b * NBLK + g * 16, 16)]
                rec = 1.0 / jnp.maximum(d, 1.0)
                dnums = lax.GatherDimensionNumbers(
                    offset_dims=(), collapsed_slice_dims=(0,),
                    start_index_map=(0,))
                for l in range(16):
                    r = g * 16 + l
                    rl = lax.gather(
                        rec, jnp.full((16, 1), l, jnp.int32), dnums,
                        slice_sizes=(1,),
                        mode=lax.GatherScatterMode.PROMISE_IN_BOUNDS)
                    for hgrp in range(HG):
                        nblk_v[r, pl.ds(hgrp * 16, 16)] = (
                            nblk_v[r, pl.ds(hgrp * 16, 16)] * rl)

            pltpu.sync_copy(nblk_v, acc_out.at[c, pl.ds(blk, NBLK)])

    return k(trans_flat, flat3d, dst3d, zacc, zdeg)


def _post(acc, x, Wself, b, g, bb):
    N, H = x.shape[0], Wself.shape[1]

    def body(acc_ref, x_ref, w_ref, b_ref, g_ref, bb_ref, o_ref):
        h = (acc_ref[0, :N] + acc_ref[1, :N]
             + jnp.dot(x_ref[...], w_ref[...],
                       preferred_element_type=jnp.float32)
             + b_ref[...])
        mu = jnp.mean(h, axis=0, keepdims=True)
        var = jnp.mean((h - mu) ** 2, axis=0, keepdims=True)
        h = (h - mu) * lax.rsqrt(var + 1e-5) * g_ref[...] + bb_ref[...]
        o_ref[...] = jnp.where(h > 0, h, jnp.exp(jnp.minimum(h, 0.0)) - 1.0)

    return pl.pallas_call(
        body,
        out_shape=jax.ShapeDtypeStruct((N, H), jnp.float32),
    )(acc, x, Wself, b, g, bb)


def _final(acc, x, Wself, b, g, bb, mW1, mb1, mg, mbb, mW2, mb2):
    N = x.shape[0]
    D_OUT = mW2.shape[1]

    def body(acc_ref, x_ref, w_ref, b_ref, g_ref, bb_ref,
             mW1_ref, mb1_ref, mg_ref, mbb_ref, mW2_ref, mb2_ref, o_ref):
        h = (acc_ref[0, :N] + acc_ref[1, :N]
             + jnp.dot(x_ref[...], w_ref[...],
                       preferred_element_type=jnp.float32)
             + b_ref[...])
        mu = jnp.mean(h, axis=0, keepdims=True)
        var = jnp.mean((h - mu) ** 2, axis=0, keepdims=True)
        h = (h - mu) * lax.rsqrt(var + 1e-5) * g_ref[...] + bb_ref[...]
        h = jnp.where(h > 0, h, jnp.exp(jnp.minimum(h, 0.0)) - 1.0)
        m = jnp.dot(h, mW1_ref[...],
                    preferred_element_type=jnp.float32) + mb1_ref[...]
        mu2 = jnp.mean(m, axis=0, keepdims=True)
        var2 = jnp.mean((m - mu2) ** 2, axis=0, keepdims=True)
        m = (m - mu2) * lax.rsqrt(var2 + 1e-5) * mg_ref[...] + mbb_ref[...]
        m = jnp.maximum(m, 0.0)
        o_ref[...] = jnp.dot(m, mW2_ref[...],
                             preferred_element_type=jnp.float32) + mb2_ref[...]

    return pl.pallas_call(
        body,
        out_shape=jax.ShapeDtypeStruct((N, D_OUT), jnp.float32),
    )(acc, x, Wself, b, g, bb, mW1, mb1, mg, mbb, mW2, mb2)


def kernel(x, edge_index0, edge_type0, edge_index1, edge_type1,
           W0, Wself0, b0, bn0_g, bn0_b,
           W1, Wself1, b1, bn1_g, bn1_b,
           mlp_W1, mlp_b1, mlp_bn_g, mlp_bn_b, mlp_W2, mlp_b2):
    N = x.shape[0]
    E = edge_type0.shape[0]
    R, _, H = W0.shape
    NPAD = ((N + 16 * NS - 1) // (16 * NS)) * (16 * NS)
    K = (E + NS * 2 * C - 1) // (NS * 2 * C)
    K = ((K + 7) // 8) * 8
    EPAD = NS * 2 * K * C

    zacc = jnp.zeros((NPAD, H), jnp.float32)
    zdeg = jnp.zeros((NPAD,), jnp.float32)

    def edges3d(edge_index, edge_type):
        pad = EPAD - E
        pad_iota = lax.iota(jnp.int32, pad)
        src = jnp.concatenate([edge_index[0], pad_iota % N])
        typ = jnp.concatenate([edge_type, jnp.zeros((pad,), jnp.int32)])
        dst = jnp.concatenate([edge_index[1], N + pad_iota % (NPAD - N)])
        flat = _flatidx(typ.reshape(EPAD // C, C), src.reshape(EPAD // C, C),
                        N)
        return flat.reshape(NS, 2 * K, C), dst.reshape(NS, 2 * K, C)

    flat0, dst0 = edges3d(edge_index0, edge_type0)
    flat1, dst1 = edges3d(edge_index1, edge_type1)

    r1h = lambda v: v.reshape(1, -1)

    trans0 = _trans(x, W0).reshape(R * N, H)
    acc0 = _sc_aggregate(trans0, flat0, dst0, zacc, zdeg, NPAD)
    h = _post(acc0, x, Wself0, r1h(b0), r1h(bn0_g), r1h(bn0_b))

    trans1 = _trans(h, W1).reshape(R * N, H)
    acc1 = _sc_aggregate(trans1, flat1, dst1, zacc, zdeg, NPAD)
    out = _final(acc1, h, Wself1, r1h(b1), r1h(bn1_g), r1h(bn1_b),
                 mlp_W1, r1h(mlp_b1), r1h(mlp_bn_g), r1h(mlp_bn_b),
                 mlp_W2, r1h(mlp_b2))
    return out


# speedup vs baseline: 35.7102x; 1.4484x over previous
import functools

import jax
import jax.numpy as jnp
from jax import lax
from jax.experimental import pallas as pl
from jax.experimental.pallas import tpu as pltpu
from jax.experimental.pallas import tpu_sc as plsc

NC = 2
NS = 16
C = 128
NBLK = 128


def _trans(x, W):
    R, D, H = W.shape
    N = x.shape[0]

    def body(x_ref, w_ref, o_ref):
        o_ref[0] = jnp.dot(x_ref[...], w_ref[0],
                           preferred_element_type=jnp.float32)

    return pl.pallas_call(
        body,
        grid=(R,),
        in_specs=[
            pl.BlockSpec((N, D), lambda r: (0, 0)),
            pl.BlockSpec((1, D, H), lambda r: (r, 0, 0)),
        ],
        out_specs=pl.BlockSpec((1, N, H), lambda r: (r, 0, 0)),
        out_shape=jax.ShapeDtypeStruct((R, N, H), jnp.float32),
    )(x, W)


def _flatidx(typ2d, src2d, n_nodes):
    def body(t_ref, s_ref, o_ref):
        o_ref[...] = t_ref[...] * n_nodes + s_ref[...]

    return pl.pallas_call(
        body,
        out_shape=jax.ShapeDtypeStruct(typ2d.shape, jnp.int32),
    )(typ2d, src2d)


def _sc_aggregate(trans_flat, flat3d, dst3d, zacc, zdeg, n_pad):
    RN, H = trans_flat.shape
    _, K2, Cc = flat3d.shape
    K = K2 // 2
    KS = K // 2
    RPW = n_pad // NS
    HG = H // 16

    mesh = plsc.VectorSubcoreMesh(core_axis_name="c", subcore_axis_name="s")

    @functools.partial(
        pl.kernel,
        mesh=mesh,
        out_type=jax.ShapeDtypeStruct((NC, n_pad, H), jnp.float32),
        scratch_types=[
            pltpu.VMEM((KS, Cc), jnp.int32),
            pltpu.VMEM((KS, Cc), jnp.int32),
            pltpu.VMEM((Cc, H), jnp.float32),
            pltpu.VMEM((Cc, H), jnp.float32),
            pltpu.VMEM((Cc,), jnp.float32),
            pltpu.VMEM((RPW,), jnp.float32),
            pltpu.SemaphoreType.DMA,
            pltpu.SemaphoreType.DMA,
            pltpu.SemaphoreType.DMA,
            pltpu.VMEM_SHARED((n_pad, H), jnp.float32),
            pltpu.VMEM_SHARED((n_pad,), jnp.float32),
        ],
    )
    def k(trans_hbm, flat_hbm, dst_hbm, zacc_hbm, zdeg_hbm, acc_out,
          flat_v, dst_v, rows_a, rows_b, ones_v, deg_v,
          sem_a, sem_b, sem_d, acc_sh, deg_sh):
        c = lax.axis_index("c")
        s = lax.axis_index("s")
        rbase = pl.multiple_of(s * RPW, 8)

        pltpu.sync_copy(zacc_hbm.at[pl.ds(rbase, RPW)],
                        acc_sh.at[pl.ds(rbase, RPW)])
        pltpu.sync_copy(zdeg_hbm.at[pl.ds(rbase, RPW)],
                        deg_sh.at[pl.ds(rbase, RPW)])

        @pl.loop(0, Cc, step=16)
        def _(i):
            ones_v[pl.ds(i, 16)] = jnp.ones((16,), jnp.float32)

        plsc.subcore_barrier()

        def gstart(j, rows_ref, sem):
            pltpu.async_copy(trans_hbm.at[flat_v.at[j]], rows_ref, sem)

        def gwait(rows_ref, sem):
            pltpu.make_async_copy(trans_hbm.at[flat_v.at[0]],
                                  rows_ref, sem).wait()

        def dstart(j):
            pltpu.async_copy(ones_v, deg_sh.at[dst_v.at[j]], sem_d,
                             add=True)

        def ddrain():
            @pl.loop(0, KS)
            def _(j):
                pltpu.make_async_copy(ones_v, deg_sh.at[pl.ds(0, Cc)],
                                      sem_d).wait()

        for t in range(2):
            base = pl.multiple_of(c * K + t * KS, 8)
            pltpu.sync_copy(flat_hbm.at[s, pl.ds(base, KS)], flat_v)
            pltpu.sync_copy(dst_hbm.at[s, pl.ds(base, KS)], dst_v)
            gstart(0, rows_a, sem_a)

            @pl.loop(0, KS // 2)
            def _(p):
                j = p * 2
                gstart(j + 1, rows_b, sem_b)
                gwait(rows_a, sem_a)
                pltpu.sync_copy(rows_a, acc_sh.at[dst_v.at[j]], add=True)
                dstart(j)

                @pl.when(j + 2 < KS)
                def _():
                    gstart(j + 2, rows_a, sem_a)

                gwait(rows_b, sem_b)
                pltpu.sync_copy(rows_b, acc_sh.at[dst_v.at[j + 1]], add=True)
                dstart(j + 1)

            ddrain()

        for t in range(2):
            base = pl.multiple_of((1 - c) * K + t * KS, 8)
            pltpu.sync_copy(dst_hbm.at[s, pl.ds(base, KS)], dst_v)

            @pl.loop(0, KS)
            def _(j):
                dstart(j)

            ddrain()

        plsc.subcore_barrier()

        pltpu.sync_copy(deg_sh.at[pl.ds(rbase, RPW)], deg_v)

        @pl.loop(0, RPW // NBLK)
        def _(b):
            blk = pl.multiple_of(rbase + b * NBLK, 8)
            pltpu.sync_copy(acc_sh.at[pl.ds(blk, NBLK)], rows_a)

            @pl.loop(0, NBLK // 16)
            def _(g):
                d = deg_v[pl.ds(b * NBLK + g * 16, 16)]
                rec = 1.0 / jnp.maximum(d, 1.0)
                dnums = lax.GatherDimensionNumbers(
                    offset_dims=(), collapsed_slice_dims=(0,),
                    start_index_map=(0,))
                for l in range(16):
                    r = g * 16 + l
                    rl = lax.gather(
                        rec, jnp.full((16, 1), l, jnp.int32), dnums,
                        slice_sizes=(1,),
                        mode=lax.GatherScatterMode.PROMISE_IN_BOUNDS)
                    for hgrp in range(HG):
                        rows_a[r, pl.ds(hgrp * 16, 16)] = (
                            rows_a[r, pl.ds(hgrp * 16, 16)] * rl)

            pltpu.sync_copy(rows_a, acc_out.at[c, pl.ds(blk, NBLK)])

    return k(trans_flat, flat3d, dst3d, zacc, zdeg)


def _post(acc, x, Wself, b, g, bb):
    N, H = x.shape[0], Wself.shape[1]

    def body(acc_ref, x_ref, w_ref, b_ref, g_ref, bb_ref, o_ref):
        h = (acc_ref[0, :N] + acc_ref[1, :N]
             + jnp.dot(x_ref[...], w_ref[...],
                       preferred_element_type=jnp.float32)
             + b_ref[...])
        mu = jnp.mean(h, axis=0, keepdims=True)
        var = jnp.mean((h - mu) ** 2, axis=0, keepdims=True)
        h = (h - mu) * lax.rsqrt(var + 1e-5) * g_ref[...] + bb_ref[...]
        o_ref[...] = jnp.where(h > 0, h, jnp.exp(jnp.minimum(h, 0.0)) - 1.0)

    return pl.pallas_call(
        body,
        out_shape=jax.ShapeDtypeStruct((N, H), jnp.float32),
    )(acc, x, Wself, b, g, bb)


def _final(acc, x, Wself, b, g, bb, mW1, mb1, mg, mbb, mW2, mb2):
    N = x.shape[0]
    D_OUT = mW2.shape[1]

    def body(acc_ref, x_ref, w_ref, b_ref, g_ref, bb_ref,
             mW1_ref, mb1_ref, mg_ref, mbb_ref, mW2_ref, mb2_ref, o_ref):
        h = (acc_ref[0, :N] + acc_ref[1, :N]
             + jnp.dot(x_ref[...], w_ref[...],
                       preferred_element_type=jnp.float32)
             + b_ref[...])
        mu = jnp.mean(h, axis=0, keepdims=True)
        var = jnp.mean((h - mu) ** 2, axis=0, keepdims=True)
        h = (h - mu) * lax.rsqrt(var + 1e-5) * g_ref[...] + bb_ref[...]
        h = jnp.where(h > 0, h, jnp.exp(jnp.minimum(h, 0.0)) - 1.0)
        m = jnp.dot(h, mW1_ref[...],
                    preferred_element_type=jnp.float32) + mb1_ref[...]
        mu2 = jnp.mean(m, axis=0, keepdims=True)
        var2 = jnp.mean((m - mu2) ** 2, axis=0, keepdims=True)
        m = (m - mu2) * lax.rsqrt(var2 + 1e-5) * mg_ref[...] + mbb_ref[...]
        m = jnp.maximum(m, 0.0)
        o_ref[...] = jnp.dot(m, mW2_ref[...],
                             preferred_element_type=jnp.float32) + mb2_ref[...]

    return pl.pallas_call(
        body,
        out_shape=jax.ShapeDtypeStruct((N, D_OUT), jnp.float32),
    )(acc, x, Wself, b, g, bb, mW1, mb1, mg, mbb, mW2, mb2)


def kernel(x, edge_index0, edge_type0, edge_index1, edge_type1,
           W0, Wself0, b0, bn0_g, bn0_b,
           W1, Wself1, b1, bn1_g, bn1_b,
           mlp_W1, mlp_b1, mlp_bn_g, mlp_bn_b, mlp_W2, mlp_b2):
    N = x.shape[0]
    E = edge_type0.shape[0]
    R, _, H = W0.shape
    NPAD = ((N + 16 * NS - 1) // (16 * NS)) * (16 * NS)
    K = (E + NS * 2 * C - 1) // (NS * 2 * C)
    K = ((K + 7) // 8) * 8
    EPAD = NS * 2 * K * C

    zacc = jnp.zeros((NPAD, H), jnp.float32)
    zdeg = jnp.zeros((NPAD,), jnp.float32)

    def edges3d(edge_index, edge_type):
        pad = EPAD - E
        pad_iota = lax.iota(jnp.int32, pad)
        src = jnp.concatenate([edge_index[0], pad_iota % N])
        typ = jnp.concatenate([edge_type, jnp.zeros((pad,), jnp.int32)])
        dst = jnp.concatenate([edge_index[1], N + pad_iota % (NPAD - N)])
        flat = _flatidx(typ.reshape(EPAD // C, C), src.reshape(EPAD // C, C),
                        N)
        return flat.reshape(NS, 2 * K, C), dst.reshape(NS, 2 * K, C)

    flat0, dst0 = edges3d(edge_index0, edge_type0)
    flat1, dst1 = edges3d(edge_index1, edge_type1)

    r1h = lambda v: v.reshape(1, -1)

    trans0 = _trans(x, W0).reshape(R * N, H)
    acc0 = _sc_aggregate(trans0, flat0, dst0, zacc, zdeg, NPAD)
    h = _post(acc0, x, Wself0, r1h(b0), r1h(bn0_g), r1h(bn0_b))

    trans1 = _trans(h, W1).reshape(R * N, H)
    acc1 = _sc_aggregate(trans1, flat1, dst1, zacc, zdeg, NPAD)
    out = _final(acc1, h, Wself1, r1h(b1), r1h(bn1_g), r1h(bn1_b),
                 mlp_W1, r1h(mlp_b1), r1h(mlp_bn_g), r1h(mlp_bn_b),
                 mlp_W2, r1h(mlp_b2))
    return out


# speedup vs baseline: 36.1335x; 1.0119x over previous
import functools

import jax
import jax.numpy as jnp
from jax import lax
from jax.experimental import pallas as pl
from jax.experimental.pallas import tpu as pltpu
from jax.experimental.pallas import tpu_sc as plsc

NC = 2
NS = 16
C = 128
NBLK = 128


def _trans(x, W, typ2d, src2d, n_nodes):
    R, D, H = W.shape
    N = x.shape[0]
    EB = typ2d.shape[0]

    def body(x_ref, w_ref, t_ref, s_ref, o_ref, f_ref):
        @pl.when(pl.program_id(0) == 0)
        def _():
            f_ref[...] = t_ref[...] * n_nodes + s_ref[...]

        o_ref[0] = jnp.dot(x_ref[...], w_ref[0],
                           preferred_element_type=jnp.float32)

    return pl.pallas_call(
        body,
        grid=(R,),
        in_specs=[
            pl.BlockSpec((N, D), lambda r: (0, 0)),
            pl.BlockSpec((1, D, H), lambda r: (r, 0, 0)),
            pl.BlockSpec((EB, 128), lambda r: (0, 0)),
            pl.BlockSpec((EB, 128), lambda r: (0, 0)),
        ],
        out_specs=[
            pl.BlockSpec((1, N, H), lambda r: (r, 0, 0)),
            pl.BlockSpec((EB, 128), lambda r: (0, 0)),
        ],
        out_shape=[
            jax.ShapeDtypeStruct((R, N, H), jnp.float32),
            jax.ShapeDtypeStruct((EB, 128), jnp.int32),
        ],
    )(x, W, typ2d, src2d)


def _post_trans(acc, x, Wself, b, g, bb, W1, typ2d, src2d, n_nodes):
    R, D, H = W1.shape
    N = x.shape[0]
    EB = typ2d.shape[0]

    def body(acc_ref, x_ref, w_ref, b_ref, g_ref, bb_ref, w1_ref,
             t_ref, s_ref, o_ref, h_ref, f_ref, hs_ref):
        @pl.when(pl.program_id(0) == 0)
        def _():
            f_ref[...] = t_ref[...] * n_nodes + s_ref[...]
            h = (acc_ref[0, :N] + acc_ref[1, :N]
                 + jnp.dot(x_ref[...], w_ref[...],
                           preferred_element_type=jnp.float32)
                 + b_ref[...])
            mu = jnp.mean(h, axis=0, keepdims=True)
            var = jnp.mean((h - mu) ** 2, axis=0, keepdims=True)
            h = (h - mu) * lax.rsqrt(var + 1e-5) * g_ref[...] + bb_ref[...]
            h = jnp.where(h > 0, h, jnp.exp(jnp.minimum(h, 0.0)) - 1.0)
            hs_ref[...] = h
            h_ref[...] = h

        o_ref[0] = jnp.dot(hs_ref[...], w1_ref[0],
                           preferred_element_type=jnp.float32)

    return pl.pallas_call(
        body,
        grid=(R,),
        in_specs=[
            pl.BlockSpec(acc.shape, lambda r: (0, 0, 0)),
            pl.BlockSpec((N, D), lambda r: (0, 0)),
            pl.BlockSpec(Wself.shape, lambda r: (0, 0)),
            pl.BlockSpec((1, H), lambda r: (0, 0)),
            pl.BlockSpec((1, H), lambda r: (0, 0)),
            pl.BlockSpec((1, H), lambda r: (0, 0)),
            pl.BlockSpec((1, D, H), lambda r: (r, 0, 0)),
            pl.BlockSpec((EB, 128), lambda r: (0, 0)),
            pl.BlockSpec((EB, 128), lambda r: (0, 0)),
        ],
        out_specs=[
            pl.BlockSpec((1, N, H), lambda r: (r, 0, 0)),
            pl.BlockSpec((N, H), lambda r: (0, 0)),
            pl.BlockSpec((EB, 128), lambda r: (0, 0)),
        ],
        out_shape=[
            jax.ShapeDtypeStruct((R, N, H), jnp.float32),
            jax.ShapeDtypeStruct((N, H), jnp.float32),
            jax.ShapeDtypeStruct((EB, 128), jnp.int32),
        ],
        scratch_shapes=[pltpu.VMEM((N, H), jnp.float32)],
    )(acc, x, Wself, b, g, bb, W1, typ2d, src2d)


def _sc_aggregate(trans_flat, flat3d, dst3d, zacc, zdeg, n_pad):
    RN, H = trans_flat.shape
    _, K2, Cc = flat3d.shape
    K = K2 // 2
    KS = K // 2
    RPW = n_pad // NS
    HG = H // 16

    mesh = plsc.VectorSubcoreMesh(core_axis_name="c", subcore_axis_name="s")

    @functools.partial(
        pl.kernel,
        mesh=mesh,
        out_type=jax.ShapeDtypeStruct((NC, n_pad, H), jnp.float32),
        scratch_types=[
            pltpu.VMEM((KS, Cc), jnp.int32),
            pltpu.VMEM((KS, Cc), jnp.int32),
            pltpu.VMEM((Cc, H), jnp.float32),
            pltpu.VMEM((Cc, H), jnp.float32),
            pltpu.VMEM((Cc,), jnp.float32),
            pltpu.VMEM((RPW,), jnp.float32),
            pltpu.SemaphoreType.DMA,
            pltpu.SemaphoreType.DMA,
            pltpu.SemaphoreType.DMA,
            pltpu.VMEM_SHARED((n_pad, H), jnp.float32),
            pltpu.VMEM_SHARED((n_pad,), jnp.float32),
        ],
    )
    def k(trans_hbm, flat_hbm, dst_hbm, zacc_hbm, zdeg_hbm, acc_out,
          flat_v, dst_v, rows_a, rows_b, ones_v, deg_v,
          sem_a, sem_b, sem_d, acc_sh, deg_sh):
        c = lax.axis_index("c")
        s = lax.axis_index("s")
        rbase = pl.multiple_of(s * RPW, 8)

        pltpu.sync_copy(zacc_hbm.at[pl.ds(rbase, RPW)],
                        acc_sh.at[pl.ds(rbase, RPW)])
        pltpu.sync_copy(zdeg_hbm.at[pl.ds(rbase, RPW)],
                        deg_sh.at[pl.ds(rbase, RPW)])

        @pl.loop(0, Cc, step=16)
        def _(i):
            ones_v[pl.ds(i, 16)] = jnp.ones((16,), jnp.float32)

        plsc.subcore_barrier()

        def gstart(j, rows_ref, sem):
            pltpu.async_copy(trans_hbm.at[flat_v.at[j]], rows_ref, sem)

        def gwait(rows_ref, sem):
            pltpu.make_async_copy(trans_hbm.at[flat_v.at[0]],
                                  rows_ref, sem).wait()

        def dstart(j):
            pltpu.async_copy(ones_v, deg_sh.at[dst_v.at[j]], sem_d,
                             add=True)

        def ddrain():
            @pl.loop(0, KS)
            def _(j):
                pltpu.make_async_copy(ones_v, deg_sh.at[pl.ds(0, Cc)],
                                      sem_d).wait()

        for t in range(2):
            base = pl.multiple_of(c * K + t * KS, 8)
            pltpu.sync_copy(flat_hbm.at[s, pl.ds(base, KS)], flat_v)
            pltpu.sync_copy(dst_hbm.at[s, pl.ds(base, KS)], dst_v)
            gstart(0, rows_a, sem_a)

            @pl.loop(0, KS // 2)
            def _(p):
                j = p * 2
                gstart(j + 1, rows_b, sem_b)
                gwait(rows_a, sem_a)
                pltpu.sync_copy(rows_a, acc_sh.at[dst_v.at[j]], add=True)
                dstart(j)

                @pl.when(j + 2 < KS)
                def _():
                    gstart(j + 2, rows_a, sem_a)

                gwait(rows_b, sem_b)
                pltpu.sync_copy(rows_b, acc_sh.at[dst_v.at[j + 1]], add=True)
                dstart(j + 1)

            ddrain()

        for t in range(2):
            base = pl.multiple_of((1 - c) * K + t * KS, 8)
            pltpu.sync_copy(dst_hbm.at[s, pl.ds(base, KS)], dst_v)

            @pl.loop(0, KS)
            def _(j):
                dstart(j)

            ddrain()

        plsc.subcore_barrier()

        pltpu.sync_copy(deg_sh.at[pl.ds(rbase, RPW)], deg_v)

        @pl.loop(0, RPW // NBLK)
        def _(b):
            blk = pl.multiple_of(rbase + b * NBLK, 8)
            pltpu.sync_copy(acc_sh.at[pl.ds(blk, NBLK)], rows_a)

            @pl.loop(0, NBLK // 16)
            def _(g):
                d = deg_v[pl.ds(b * NBLK + g * 16, 16)]
                rec = 1.0 / jnp.maximum(d, 1.0)
                dnums = lax.GatherDimensionNumbers(
                    offset_dims=(), collapsed_slice_dims=(0,),
                    start_index_map=(0,))
                for l in range(16):
                    r = g * 16 + l
                    rl = lax.gather(
                        rec, jnp.full((16, 1), l, jnp.int32), dnums,
                        slice_sizes=(1,),
                        mode=lax.GatherScatterMode.PROMISE_IN_BOUNDS)
                    for hgrp in range(HG):
                        rows_a[r, pl.ds(hgrp * 16, 16)] = (
                            rows_a[r, pl.ds(hgrp * 16, 16)] * rl)

            pltpu.sync_copy(rows_a, acc_out.at[c, pl.ds(blk, NBLK)])

    return k(trans_flat, flat3d, dst3d, zacc, zdeg)


def _final(acc, x, Wself, b, g, bb, mW1, mb1, mg, mbb, mW2, mb2):
    N = x.shape[0]
    D_OUT = mW2.shape[1]

    def body(acc_ref, x_ref, w_ref, b_ref, g_ref, bb_ref,
             mW1_ref, mb1_ref, mg_ref, mbb_ref, mW2_ref, mb2_ref, o_ref):
        h = (acc_ref[0, :N] + acc_ref[1, :N]
             + jnp.dot(x_ref[...], w_ref[...],
                       preferred_element_type=jnp.float32)
             + b_ref[...])
        mu = jnp.mean(h, axis=0, keepdims=True)
        var = jnp.mean((h - mu) ** 2, axis=0, keepdims=True)
        h = (h - mu) * lax.rsqrt(var + 1e-5) * g_ref[...] + bb_ref[...]
        h = jnp.where(h > 0, h, jnp.exp(jnp.minimum(h, 0.0)) - 1.0)
        m = jnp.dot(h, mW1_ref[...],
                    preferred_element_type=jnp.float32) + mb1_ref[...]
        mu2 = jnp.mean(m, axis=0, keepdims=True)
        var2 = jnp.mean((m - mu2) ** 2, axis=0, keepdims=True)
        m = (m - mu2) * lax.rsqrt(var2 + 1e-5) * mg_ref[...] + mbb_ref[...]
        m = jnp.maximum(m, 0.0)
        o_ref[...] = jnp.dot(m, mW2_ref[...],
                             preferred_element_type=jnp.float32) + mb2_ref[...]

    return pl.pallas_call(
        body,
        out_shape=jax.ShapeDtypeStruct((N, D_OUT), jnp.float32),
    )(acc, x, Wself, b, g, bb, mW1, mb1, mg, mbb, mW2, mb2)


def kernel(x, edge_index0, edge_type0, edge_index1, edge_type1,
           W0, Wself0, b0, bn0_g, bn0_b,
           W1, Wself1, b1, bn1_g, bn1_b,
           mlp_W1, mlp_b1, mlp_bn_g, mlp_bn_b, mlp_W2, mlp_b2):
    N = x.shape[0]
    E = edge_type0.shape[0]
    R, _, H = W0.shape
    NPAD = ((N + 16 * NS - 1) // (16 * NS)) * (16 * NS)
    K = (E + NS * 2 * C - 1) // (NS * 2 * C)
    K = ((K + 7) // 8) * 8
    EPAD = NS * 2 * K * C

    zacc = jnp.zeros((NPAD, H), jnp.float32)
    zdeg = jnp.zeros((NPAD,), jnp.float32)

    def edges_prep(edge_index, edge_type):
        pad = EPAD - E
        pad_iota = lax.iota(jnp.int32, pad)
        src = jnp.concatenate([edge_index[0], pad_iota % N])
        typ = jnp.concatenate([edge_type, jnp.zeros((pad,), jnp.int32)])
        dst = jnp.concatenate([edge_index[1], N + pad_iota % (NPAD - N)])
        return (typ.reshape(EPAD // C, C), src.reshape(EPAD // C, C),
                dst.reshape(NS, 2 * K, C))

    typ0_2d, src0_2d, dst0 = edges_prep(edge_index0, edge_type0)
    typ1_2d, src1_2d, dst1 = edges_prep(edge_index1, edge_type1)

    r1h = lambda v: v.reshape(1, -1)

    trans0, flat0 = _trans(x, W0, typ0_2d, src0_2d, N)
    acc0 = _sc_aggregate(trans0.reshape(R * N, H),
                         flat0.reshape(NS, 2 * K, C), dst0, zacc, zdeg, NPAD)
    trans1, h, flat1 = _post_trans(acc0, x, Wself0, r1h(b0), r1h(bn0_g),
                                   r1h(bn0_b), W1, typ1_2d, src1_2d, N)
    acc1 = _sc_aggregate(trans1.reshape(R * N, H),
                         flat1.reshape(NS, 2 * K, C), dst1, zacc, zdeg, NPAD)
    out = _final(acc1, h, Wself1, r1h(b1), r1h(bn1_g), r1h(bn1_b),
                 mlp_W1, r1h(mlp_b1), r1h(mlp_bn_g), r1h(mlp_bn_b),
                 mlp_W2, r1h(mlp_b2))
    return out


# speedup vs baseline: 37.2029x; 1.0296x over previous
import functools

import jax
import jax.numpy as jnp
from jax import lax
from jax.experimental import pallas as pl
from jax.experimental.pallas import tpu as pltpu
from jax.experimental.pallas import tpu_sc as plsc

NC = 2
NS = 16
C = 128
NBLK = 128


def _trans(x, W, typ2d, src2d, n_nodes):
    R, D, H = W.shape
    N = x.shape[0]
    EB = typ2d.shape[0]

    def body(x_ref, w_ref, t_ref, s_ref, o_ref, f_ref):
        @pl.when(pl.program_id(0) == 0)
        def _():
            f_ref[...] = t_ref[...] * n_nodes + s_ref[...]

        o_ref[0] = jnp.dot(x_ref[...], w_ref[0],
                           preferred_element_type=jnp.float32)

    return pl.pallas_call(
        body,
        grid=(R,),
        in_specs=[
            pl.BlockSpec((N, D), lambda r: (0, 0)),
            pl.BlockSpec((1, D, H), lambda r: (r, 0, 0)),
            pl.BlockSpec((EB, 128), lambda r: (0, 0)),
            pl.BlockSpec((EB, 128), lambda r: (0, 0)),
        ],
        out_specs=[
            pl.BlockSpec((1, N, H), lambda r: (r, 0, 0)),
            pl.BlockSpec((EB, 128), lambda r: (0, 0)),
        ],
        out_shape=[
            jax.ShapeDtypeStruct((R, N, H), jnp.float32),
            jax.ShapeDtypeStruct((EB, 128), jnp.int32),
        ],
    )(x, W, typ2d, src2d)


def _post_trans(acc, x, Wself, b, g, bb, W1, typ2d, src2d, n_nodes):
    R, D, H = W1.shape
    N = x.shape[0]
    EB = typ2d.shape[0]

    def body(acc_ref, x_ref, w_ref, b_ref, g_ref, bb_ref, w1_ref,
             t_ref, s_ref, o_ref, h_ref, f_ref, hs_ref):
        @pl.when(pl.program_id(0) == 0)
        def _():
            f_ref[...] = t_ref[...] * n_nodes + s_ref[...]
            h = (acc_ref[0, :N] + acc_ref[1, :N]
                 + jnp.dot(x_ref[...], w_ref[...],
                           preferred_element_type=jnp.float32)
                 + b_ref[...])
            mu = jnp.mean(h, axis=0, keepdims=True)
            var = jnp.mean((h - mu) ** 2, axis=0, keepdims=True)
            h = (h - mu) * lax.rsqrt(var + 1e-5) * g_ref[...] + bb_ref[...]
            h = jnp.where(h > 0, h, jnp.exp(jnp.minimum(h, 0.0)) - 1.0)
            hs_ref[...] = h
            h_ref[...] = h

        o_ref[0] = jnp.dot(hs_ref[...], w1_ref[0],
                           preferred_element_type=jnp.float32)

    return pl.pallas_call(
        body,
        grid=(R,),
        in_specs=[
            pl.BlockSpec(acc.shape, lambda r: (0, 0, 0)),
            pl.BlockSpec((N, D), lambda r: (0, 0)),
            pl.BlockSpec(Wself.shape, lambda r: (0, 0)),
            pl.BlockSpec((1, H), lambda r: (0, 0)),
            pl.BlockSpec((1, H), lambda r: (0, 0)),
            pl.BlockSpec((1, H), lambda r: (0, 0)),
            pl.BlockSpec((1, D, H), lambda r: (r, 0, 0)),
            pl.BlockSpec((EB, 128), lambda r: (0, 0)),
            pl.BlockSpec((EB, 128), lambda r: (0, 0)),
        ],
        out_specs=[
            pl.BlockSpec((1, N, H), lambda r: (r, 0, 0)),
            pl.BlockSpec((N, H), lambda r: (0, 0)),
            pl.BlockSpec((EB, 128), lambda r: (0, 0)),
        ],
        out_shape=[
            jax.ShapeDtypeStruct((R, N, H), jnp.float32),
            jax.ShapeDtypeStruct((N, H), jnp.float32),
            jax.ShapeDtypeStruct((EB, 128), jnp.int32),
        ],
        scratch_shapes=[pltpu.VMEM((N, H), jnp.float32)],
    )(acc, x, Wself, b, g, bb, W1, typ2d, src2d)


def _sc_degree(dst3d, zdeg, n_pad):
    _, K2, Cc = dst3d.shape
    KS = K2 // 4
    RPW = n_pad // NS

    mesh = plsc.VectorSubcoreMesh(core_axis_name="c", subcore_axis_name="s")

    @functools.partial(
        pl.kernel,
        mesh=mesh,
        out_type=jax.ShapeDtypeStruct((NC * n_pad,), jnp.float32),
        scratch_types=[
            pltpu.VMEM((KS, Cc), jnp.int32),
            pltpu.VMEM((Cc,), jnp.float32),
            pltpu.SemaphoreType.DMA,
            pltpu.VMEM_SHARED((n_pad,), jnp.float32),
        ],
    )
    def k(dst_hbm, zdeg_hbm, deg_out, dst_v, ones_v, sem_d, deg_sh):
        c = lax.axis_index("c")
        s = lax.axis_index("s")
        rbase = pl.multiple_of(s * RPW, 8)

        pltpu.sync_copy(zdeg_hbm.at[pl.ds(rbase, RPW)],
                        deg_sh.at[pl.ds(rbase, RPW)])

        @pl.loop(0, Cc, step=16)
        def _(i):
            ones_v[pl.ds(i, 16)] = jnp.ones((16,), jnp.float32)

        plsc.subcore_barrier()

        for t in range(4):
            base = pl.multiple_of(t * KS, 8)
            pltpu.sync_copy(dst_hbm.at[s, pl.ds(base, KS)], dst_v)

            @pl.loop(0, KS)
            def _(j):
                pltpu.async_copy(ones_v, deg_sh.at[dst_v.at[j]], sem_d,
                                 add=True)

            @pl.loop(0, KS)
            def _(j):
                pltpu.make_async_copy(ones_v, deg_sh.at[pl.ds(0, Cc)],
                                      sem_d).wait()

        plsc.subcore_barrier()

        out_base = pl.multiple_of(c * n_pad + s * RPW, 8)
        pltpu.sync_copy(deg_sh.at[pl.ds(rbase, RPW)],
                        deg_out.at[pl.ds(out_base, RPW)])

    return k(dst3d, zdeg)


def _sc_aggregate(trans_flat, flat3d, dst3d, deg1d, zacc, n_pad):
    RN, H = trans_flat.shape
    _, K2, Cc = flat3d.shape
    K = K2 // 2
    KS = K // 2
    RPW = n_pad // NS
    HG = H // 16

    mesh = plsc.VectorSubcoreMesh(core_axis_name="c", subcore_axis_name="s")

    @functools.partial(
        pl.kernel,
        mesh=mesh,
        out_type=jax.ShapeDtypeStruct((NC, n_pad, H), jnp.float32),
        scratch_types=[
            pltpu.VMEM((KS, Cc), jnp.int32),
            pltpu.VMEM((KS, Cc), jnp.int32),
            pltpu.VMEM((Cc, H), jnp.float32),
            pltpu.VMEM((Cc, H), jnp.float32),
            pltpu.VMEM((RPW,), jnp.float32),
            pltpu.SemaphoreType.DMA,
            pltpu.SemaphoreType.DMA,
            pltpu.VMEM_SHARED((n_pad, H), jnp.float32),
        ],
    )
    def k(trans_hbm, flat_hbm, dst_hbm, deg_hbm, zacc_hbm, acc_out,
          flat_v, dst_v, rows_a, rows_b, deg_v, sem_a, sem_b, acc_sh):
        c = lax.axis_index("c")
        s = lax.axis_index("s")
        rbase = pl.multiple_of(s * RPW, 8)

        pltpu.sync_copy(zacc_hbm.at[pl.ds(rbase, RPW)],
                        acc_sh.at[pl.ds(rbase, RPW)])

        pltpu.sync_copy(deg_hbm.at[pl.ds(pl.multiple_of(
            c * n_pad + s * RPW, 8), RPW)], deg_v)

        plsc.subcore_barrier()

        def gstart(j, rows_ref, sem):
            pltpu.async_copy(trans_hbm.at[flat_v.at[j]], rows_ref, sem)

        def gwait(rows_ref, sem):
            pltpu.make_async_copy(trans_hbm.at[flat_v.at[0]],
                                  rows_ref, sem).wait()

        for t in range(2):
            base = pl.multiple_of(c * K + t * KS, 8)
            pltpu.sync_copy(flat_hbm.at[s, pl.ds(base, KS)], flat_v)
            pltpu.sync_copy(dst_hbm.at[s, pl.ds(base, KS)], dst_v)
            gstart(0, rows_a, sem_a)

            @pl.loop(0, KS // 2)
            def _(p):
                j = p * 2
                gstart(j + 1, rows_b, sem_b)
                gwait(rows_a, sem_a)
                pltpu.sync_copy(rows_a, acc_sh.at[dst_v.at[j]], add=True)

                @pl.when(j + 2 < KS)
                def _():
                    gstart(j + 2, rows_a, sem_a)

                gwait(rows_b, sem_b)
                pltpu.sync_copy(rows_b, acc_sh.at[dst_v.at[j + 1]], add=True)

        plsc.subcore_barrier()


        @pl.loop(0, RPW // NBLK)
        def _(b):
            blk = pl.multiple_of(rbase + b * NBLK, 8)
            pltpu.sync_copy(acc_sh.at[pl.ds(blk, NBLK)], rows_a)

            @pl.loop(0, NBLK // 16)
            def _(g):
                d = deg_v[pl.ds(b * NBLK + g * 16, 16)]
                rec = 1.0 / jnp.maximum(d, 1.0)
                dnums = lax.GatherDimensionNumbers(
                    offset_dims=(), collapsed_slice_dims=(0,),
                    start_index_map=(0,))
                for l in range(16):
                    r = g * 16 + l
                    rl = lax.gather(
                        rec, jnp.full((16, 1), l, jnp.int32), dnums,
                        slice_sizes=(1,),
                        mode=lax.GatherScatterMode.PROMISE_IN_BOUNDS)
                    for hgrp in range(HG):
                        rows_a[r, pl.ds(hgrp * 16, 16)] = (
                            rows_a[r, pl.ds(hgrp * 16, 16)] * rl)

            pltpu.sync_copy(rows_a, acc_out.at[c, pl.ds(blk, NBLK)])

    return k(trans_flat, flat3d, dst3d, deg1d, zacc)


def _final(acc, x, Wself, b, g, bb, mW1, mb1, mg, mbb, mW2, mb2):
    N = x.shape[0]
    D_OUT = mW2.shape[1]

    def body(acc_ref, x_ref, w_ref, b_ref, g_ref, bb_ref,
             mW1_ref, mb1_ref, mg_ref, mbb_ref, mW2_ref, mb2_ref, o_ref):
        h = (acc_ref[0, :N] + acc_ref[1, :N]
             + jnp.dot(x_ref[...], w_ref[...],
                       preferred_element_type=jnp.float32)
             + b_ref[...])
        mu = jnp.mean(h, axis=0, keepdims=True)
        var = jnp.mean((h - mu) ** 2, axis=0, keepdims=True)
        h = (h - mu) * lax.rsqrt(var + 1e-5) * g_ref[...] + bb_ref[...]
        h = jnp.where(h > 0, h, jnp.exp(jnp.minimum(h, 0.0)) - 1.0)
        m = jnp.dot(h, mW1_ref[...],
                    preferred_element_type=jnp.float32) + mb1_ref[...]
        mu2 = jnp.mean(m, axis=0, keepdims=True)
        var2 = jnp.mean((m - mu2) ** 2, axis=0, keepdims=True)
        m = (m - mu2) * lax.rsqrt(var2 + 1e-5) * mg_ref[...] + mbb_ref[...]
        m = jnp.maximum(m, 0.0)
        o_ref[...] = jnp.dot(m, mW2_ref[...],
                             preferred_element_type=jnp.float32) + mb2_ref[...]

    return pl.pallas_call(
        body,
        out_shape=jax.ShapeDtypeStruct((N, D_OUT), jnp.float32),
    )(acc, x, Wself, b, g, bb, mW1, mb1, mg, mbb, mW2, mb2)


def kernel(x, edge_index0, edge_type0, edge_index1, edge_type1,
           W0, Wself0, b0, bn0_g, bn0_b,
           W1, Wself1, b1, bn1_g, bn1_b,
           mlp_W1, mlp_b1, mlp_bn_g, mlp_bn_b, mlp_W2, mlp_b2):
    N = x.shape[0]
    E = edge_type0.shape[0]
    R, _, H = W0.shape
    NPAD = ((N + 16 * NS - 1) // (16 * NS)) * (16 * NS)
    K = (E + NS * 2 * C - 1) // (NS * 2 * C)
    K = ((K + 7) // 8) * 8
    EPAD = NS * 2 * K * C

    zacc = jnp.zeros((NPAD, H), jnp.float32)
    zdeg = jnp.zeros((NPAD,), jnp.float32)

    def edges_prep(edge_index, edge_type):
        pad = EPAD - E
        pad_iota = lax.iota(jnp.int32, pad)
        src = jnp.concatenate([edge_index[0], pad_iota % N])
        typ = jnp.concatenate([edge_type, jnp.zeros((pad,), jnp.int32)])
        dst = jnp.concatenate([edge_index[1], N + pad_iota % (NPAD - N)])
        return (typ.reshape(EPAD // C, C), src.reshape(EPAD // C, C),
                dst.reshape(NS, 2 * K, C))

    typ0_2d, src0_2d, dst0 = edges_prep(edge_index0, edge_type0)
    typ1_2d, src1_2d, dst1 = edges_prep(edge_index1, edge_type1)

    r1h = lambda v: v.reshape(1, -1)

    deg0 = _sc_degree(dst0, zdeg, NPAD)
    trans0, flat0 = _trans(x, W0, typ0_2d, src0_2d, N)
    acc0 = _sc_aggregate(trans0.reshape(R * N, H),
                         flat0.reshape(NS, 2 * K, C), dst0, deg0, zacc, NPAD)
    deg1 = _sc_degree(dst1, zdeg, NPAD)
    trans1, h, flat1 = _post_trans(acc0, x, Wself0, r1h(b0), r1h(bn0_g),
                                   r1h(bn0_b), W1, typ1_2d, src1_2d, N)
    acc1 = _sc_aggregate(trans1.reshape(R * N, H),
                         flat1.reshape(NS, 2 * K, C), dst1, deg1, zacc, NPAD)
    out = _final(acc1, h, Wself1, r1h(b1), r1h(bn1_g), r1h(bn1_b),
                 mlp_W1, r1h(mlp_b1), r1h(mlp_bn_g), r1h(mlp_bn_b),
                 mlp_W2, r1h(mlp_b2))
    return out


# speedup vs baseline: 37.7436x; 1.0145x over previous
import functools

import jax
import jax.numpy as jnp
from jax import lax
from jax.experimental import pallas as pl
from jax.experimental.pallas import tpu as pltpu
from jax.experimental.pallas import tpu_sc as plsc

NC = 2
NS = 16
C = 128
NBLK = 128


def _trans(x, W, typ2d, src2d, n_nodes):
    R, D, H = W.shape
    N = x.shape[0]
    EB = typ2d.shape[0]

    def body(x_ref, w_ref, t_ref, s_ref, o_ref, f_ref):
        @pl.when(pl.program_id(0) == 0)
        def _():
            f_ref[...] = t_ref[...] * n_nodes + s_ref[...]

        o_ref[0] = jnp.dot(x_ref[...], w_ref[0],
                           preferred_element_type=jnp.float32)

    return pl.pallas_call(
        body,
        grid=(R,),
        in_specs=[
            pl.BlockSpec((N, D), lambda r: (0, 0)),
            pl.BlockSpec((1, D, H), lambda r: (r, 0, 0)),
            pl.BlockSpec((EB, 128), lambda r: (0, 0)),
            pl.BlockSpec((EB, 128), lambda r: (0, 0)),
        ],
        out_specs=[
            pl.BlockSpec((1, N, H), lambda r: (r, 0, 0)),
            pl.BlockSpec((EB, 128), lambda r: (0, 0)),
        ],
        out_shape=[
            jax.ShapeDtypeStruct((R, N, H), jnp.float32),
            jax.ShapeDtypeStruct((EB, 128), jnp.int32),
        ],
    )(x, W, typ2d, src2d)


def _post_trans(acc, x, Wself, b, g, bb, W1, typ2d, src2d, n_nodes):
    R, D, H = W1.shape
    N = x.shape[0]
    EB = typ2d.shape[0]

    def body(acc_ref, x_ref, w_ref, b_ref, g_ref, bb_ref, w1_ref,
             t_ref, s_ref, o_ref, h_ref, f_ref, hs_ref):
        @pl.when(pl.program_id(0) == 0)
        def _():
            f_ref[...] = t_ref[...] * n_nodes + s_ref[...]
            h = (acc_ref[0, :N] + acc_ref[1, :N]
                 + jnp.dot(x_ref[...], w_ref[...],
                           preferred_element_type=jnp.float32)
                 + b_ref[...])
            mu = jnp.mean(h, axis=0, keepdims=True)
            var = jnp.mean((h - mu) ** 2, axis=0, keepdims=True)
            h = (h - mu) * lax.rsqrt(var + 1e-5) * g_ref[...] + bb_ref[...]
            h = jnp.where(h > 0, h, jnp.exp(jnp.minimum(h, 0.0)) - 1.0)
            hs_ref[...] = h
            h_ref[...] = h

        o_ref[0] = jnp.dot(hs_ref[...], w1_ref[0],
                           preferred_element_type=jnp.float32)

    return pl.pallas_call(
        body,
        grid=(R,),
        in_specs=[
            pl.BlockSpec(acc.shape, lambda r: (0, 0, 0)),
            pl.BlockSpec((N, D), lambda r: (0, 0)),
            pl.BlockSpec(Wself.shape, lambda r: (0, 0)),
            pl.BlockSpec((1, H), lambda r: (0, 0)),
            pl.BlockSpec((1, H), lambda r: (0, 0)),
            pl.BlockSpec((1, H), lambda r: (0, 0)),
            pl.BlockSpec((1, D, H), lambda r: (r, 0, 0)),
            pl.BlockSpec((EB, 128), lambda r: (0, 0)),
            pl.BlockSpec((EB, 128), lambda r: (0, 0)),
        ],
        out_specs=[
            pl.BlockSpec((1, N, H), lambda r: (r, 0, 0)),
            pl.BlockSpec((N, H), lambda r: (0, 0)),
            pl.BlockSpec((EB, 128), lambda r: (0, 0)),
        ],
        out_shape=[
            jax.ShapeDtypeStruct((R, N, H), jnp.float32),
            jax.ShapeDtypeStruct((N, H), jnp.float32),
            jax.ShapeDtypeStruct((EB, 128), jnp.int32),
        ],
        scratch_shapes=[pltpu.VMEM((N, H), jnp.float32)],
    )(acc, x, Wself, b, g, bb, W1, typ2d, src2d)


def _sc_degree(dst3d, zdeg, n_pad):
    _, K2, Cc = dst3d.shape
    KS = K2 // 4
    RPW = n_pad // NS

    mesh = plsc.VectorSubcoreMesh(core_axis_name="c", subcore_axis_name="s")

    @functools.partial(
        pl.kernel,
        mesh=mesh,
        out_type=jax.ShapeDtypeStruct((NC * n_pad,), jnp.float32),
        scratch_types=[
            pltpu.VMEM((KS, Cc), jnp.int32),
            pltpu.VMEM((Cc,), jnp.float32),
            pltpu.SemaphoreType.DMA,
            pltpu.VMEM_SHARED((n_pad,), jnp.float32),
        ],
    )
    def k(dst_hbm, zdeg_hbm, deg_out, dst_v, ones_v, sem_d, deg_sh):
        c = lax.axis_index("c")
        s = lax.axis_index("s")
        rbase = pl.multiple_of(s * RPW, 8)

        pltpu.sync_copy(zdeg_hbm.at[pl.ds(rbase, RPW)],
                        deg_sh.at[pl.ds(rbase, RPW)])

        @pl.loop(0, Cc, step=16)
        def _(i):
            ones_v[pl.ds(i, 16)] = jnp.ones((16,), jnp.float32)

        plsc.subcore_barrier()

        for t in range(4):
            base = pl.multiple_of(t * KS, 8)
            pltpu.sync_copy(dst_hbm.at[s, pl.ds(base, KS)], dst_v)

            @pl.loop(0, KS)
            def _(j):
                pltpu.async_copy(ones_v, deg_sh.at[dst_v.at[j]], sem_d,
                                 add=True)

            @pl.loop(0, KS)
            def _(j):
                pltpu.make_async_copy(ones_v, deg_sh.at[pl.ds(0, Cc)],
                                      sem_d).wait()

        plsc.subcore_barrier()

        out_base = pl.multiple_of(c * n_pad + s * RPW, 8)
        pltpu.sync_copy(deg_sh.at[pl.ds(rbase, RPW)],
                        deg_out.at[pl.ds(out_base, RPW)])

    return k(dst3d, zdeg)


def _sc_aggregate(trans_flat, flat3d, dst3d, deg1d, zacc, n_pad):
    RN, H = trans_flat.shape
    _, K2, Cc = flat3d.shape
    K = K2 // 2
    KS = K // 2
    RPW = n_pad // NS
    HG = H // 16

    mesh = plsc.VectorSubcoreMesh(core_axis_name="c", subcore_axis_name="s")

    @functools.partial(
        pl.kernel,
        mesh=mesh,
        out_type=jax.ShapeDtypeStruct((NC, n_pad, H), jnp.float32),
        scratch_types=[
            pltpu.VMEM((KS, Cc), jnp.int32),
            pltpu.VMEM((KS, Cc), jnp.int32),
            pltpu.VMEM((Cc, H), jnp.float32),
            pltpu.VMEM((Cc, H), jnp.float32),
            pltpu.VMEM((RPW,), jnp.float32),
            pltpu.SemaphoreType.DMA,
            pltpu.SemaphoreType.DMA,
            pltpu.SemaphoreType.DMA,
            pltpu.SemaphoreType.DMA,
            pltpu.VMEM_SHARED((n_pad, H), jnp.float32),
        ],
    )
    def k(trans_hbm, flat_hbm, dst_hbm, deg_hbm, zacc_hbm, acc_out,
          flat_v, dst_v, rows_a, rows_b, deg_v, sem_a, sem_b,
          sem_oa, sem_ob, acc_sh):
        c = lax.axis_index("c")
        s = lax.axis_index("s")
        rbase = pl.multiple_of(s * RPW, 8)

        pltpu.sync_copy(zacc_hbm.at[pl.ds(rbase, RPW)],
                        acc_sh.at[pl.ds(rbase, RPW)])

        pltpu.sync_copy(deg_hbm.at[pl.ds(pl.multiple_of(
            c * n_pad + s * RPW, 8), RPW)], deg_v)

        plsc.subcore_barrier()

        def gstart(j, rows_ref, sem):
            pltpu.async_copy(trans_hbm.at[flat_v.at[j]], rows_ref, sem)

        def gwait(rows_ref, sem):
            pltpu.make_async_copy(trans_hbm.at[flat_v.at[0]],
                                  rows_ref, sem).wait()

        for t in range(2):
            base = pl.multiple_of(c * K + t * KS, 8)
            pltpu.sync_copy(flat_hbm.at[s, pl.ds(base, KS)], flat_v)
            pltpu.sync_copy(dst_hbm.at[s, pl.ds(base, KS)], dst_v)
            gstart(0, rows_a, sem_a)

            @pl.loop(0, KS // 2)
            def _(p):
                j = p * 2
                gstart(j + 1, rows_b, sem_b)
                gwait(rows_a, sem_a)
                pltpu.sync_copy(rows_a, acc_sh.at[dst_v.at[j]], add=True)

                @pl.when(j + 2 < KS)
                def _():
                    gstart(j + 2, rows_a, sem_a)

                gwait(rows_b, sem_b)
                pltpu.sync_copy(rows_b, acc_sh.at[dst_v.at[j + 1]], add=True)

        plsc.subcore_barrier()

        NB = RPW // NBLK
        bufs = [rows_a, rows_b]
        isems = [sem_a, sem_b]
        osems = [sem_oa, sem_ob]

        def blk_off(b):
            return pl.multiple_of(rbase + b * NBLK, 8)

        def norm_block(buf, b):
            @pl.loop(0, NBLK // 16)
            def _(g):
                d = deg_v[pl.ds(b * NBLK + g * 16, 16)]
                rec = 1.0 / jnp.maximum(d, 1.0)
                dnums = lax.GatherDimensionNumbers(
                    offset_dims=(), collapsed_slice_dims=(0,),
                    start_index_map=(0,))
                for l in range(16):
                    r = g * 16 + l
                    rl = lax.gather(
                        rec, jnp.full((16, 1), l, jnp.int32), dnums,
                        slice_sizes=(1,),
                        mode=lax.GatherScatterMode.PROMISE_IN_BOUNDS)
                    for hgrp in range(HG):
                        buf[r, pl.ds(hgrp * 16, 16)] = (
                            buf[r, pl.ds(hgrp * 16, 16)] * rl)

        pltpu.async_copy(acc_sh.at[pl.ds(blk_off(0), NBLK)], rows_a, sem_a)
        for b in range(NB):
            buf, si, so = bufs[b % 2], isems[b % 2], osems[b % 2]
            pltpu.make_async_copy(acc_sh.at[pl.ds(0, NBLK)], buf, si).wait()
            if b + 1 < NB:
                nbuf = bufs[(b + 1) % 2]
                if b >= 1:
                    pltpu.make_async_copy(
                        nbuf, acc_out.at[c, pl.ds(rbase, NBLK)],
                        osems[(b + 1) % 2]).wait()
                pltpu.async_copy(acc_sh.at[pl.ds(blk_off(b + 1), NBLK)],
                                 nbuf, isems[(b + 1) % 2])
            norm_block(buf, b)
            pltpu.async_copy(buf, acc_out.at[c, pl.ds(blk_off(b), NBLK)], so)

        pltpu.make_async_copy(bufs[(NB - 1) % 2],
                              acc_out.at[c, pl.ds(rbase, NBLK)],
                              osems[(NB - 1) % 2]).wait()

    return k(trans_flat, flat3d, dst3d, deg1d, zacc)


def _final(acc, x, Wself, b, g, bb, mW1, mb1, mg, mbb, mW2, mb2):
    N = x.shape[0]
    D_OUT = mW2.shape[1]

    def body(acc_ref, x_ref, w_ref, b_ref, g_ref, bb_ref,
             mW1_ref, mb1_ref, mg_ref, mbb_ref, mW2_ref, mb2_ref, o_ref):
        h = (acc_ref[0, :N] + acc_ref[1, :N]
             + jnp.dot(x_ref[...], w_ref[...],
                       preferred_element_type=jnp.float32)
             + b_ref[...])
        mu = jnp.mean(h, axis=0, keepdims=True)
        var = jnp.mean((h - mu) ** 2, axis=0, keepdims=True)
        h = (h - mu) * lax.rsqrt(var + 1e-5) * g_ref[...] + bb_ref[...]
        h = jnp.where(h > 0, h, jnp.exp(jnp.minimum(h, 0.0)) - 1.0)
        m = jnp.dot(h, mW1_ref[...],
                    preferred_element_type=jnp.float32) + mb1_ref[...]
        mu2 = jnp.mean(m, axis=0, keepdims=True)
        var2 = jnp.mean((m - mu2) ** 2, axis=0, keepdims=True)
        m = (m - mu2) * lax.rsqrt(var2 + 1e-5) * mg_ref[...] + mbb_ref[...]
        m = jnp.maximum(m, 0.0)
        o_ref[...] = jnp.dot(m, mW2_ref[...],
                             preferred_element_type=jnp.float32) + mb2_ref[...]

    return pl.pallas_call(
        body,
        out_shape=jax.ShapeDtypeStruct((N, D_OUT), jnp.float32),
    )(acc, x, Wself, b, g, bb, mW1, mb1, mg, mbb, mW2, mb2)


def kernel(x, edge_index0, edge_type0, edge_index1, edge_type1,
           W0, Wself0, b0, bn0_g, bn0_b,
           W1, Wself1, b1, bn1_g, bn1_b,
           mlp_W1, mlp_b1, mlp_bn_g, mlp_bn_b, mlp_W2, mlp_b2):
    N = x.shape[0]
    E = edge_type0.shape[0]
    R, _, H = W0.shape
    NPAD = ((N + 16 * NS - 1) // (16 * NS)) * (16 * NS)
    K = (E + NS * 2 * C - 1) // (NS * 2 * C)
    K = ((K + 7) // 8) * 8
    EPAD = NS * 2 * K * C

    zacc = jnp.zeros((NPAD, H), jnp.float32)
    zdeg = jnp.zeros((NPAD,), jnp.float32)

    def edges_prep(edge_index, edge_type):
        pad = EPAD - E
        pad_iota = lax.iota(jnp.int32, pad)
        src = jnp.concatenate([edge_index[0], pad_iota % N])
        typ = jnp.concatenate([edge_type, jnp.zeros((pad,), jnp.int32)])
        dst = jnp.concatenate([edge_index[1], N + pad_iota % (NPAD - N)])
        return (typ.reshape(EPAD // C, C), src.reshape(EPAD // C, C),
                dst.reshape(NS, 2 * K, C))

    typ0_2d, src0_2d, dst0 = edges_prep(edge_index0, edge_type0)
    typ1_2d, src1_2d, dst1 = edges_prep(edge_index1, edge_type1)

    r1h = lambda v: v.reshape(1, -1)

    deg0 = _sc_degree(dst0, zdeg, NPAD)
    trans0, flat0 = _trans(x, W0, typ0_2d, src0_2d, N)
    acc0 = _sc_aggregate(trans0.reshape(R * N, H),
                         flat0.reshape(NS, 2 * K, C), dst0, deg0, zacc, NPAD)
    deg1 = _sc_degree(dst1, zdeg, NPAD)
    trans1, h, flat1 = _post_trans(acc0, x, Wself0, r1h(b0), r1h(bn0_g),
                                   r1h(bn0_b), W1, typ1_2d, src1_2d, N)
    acc1 = _sc_aggregate(trans1.reshape(R * N, H),
                         flat1.reshape(NS, 2 * K, C), dst1, deg1, zacc, NPAD)
    out = _final(acc1, h, Wself1, r1h(b1), r1h(bn1_g), r1h(bn1_b),
                 mlp_W1, r1h(mlp_b1), r1h(mlp_bn_g), r1h(mlp_bn_b),
                 mlp_W2, r1h(mlp_b2))
    return out


# speedup vs baseline: 37.8028x; 1.0016x over previous
import functools

import jax
import jax.numpy as jnp
from jax import lax
from jax.experimental import pallas as pl
from jax.experimental.pallas import tpu as pltpu
from jax.experimental.pallas import tpu_sc as plsc

NC = 2
NS = 16
C = 128
NBLK = 128


def _trans(x, W, typ2d, src2d, n_nodes):
    R, D, H = W.shape
    N = x.shape[0]
    EB = typ2d.shape[0]

    def body(x_ref, w_ref, t_ref, s_ref, o_ref, f_ref):
        @pl.when(pl.program_id(0) == 0)
        def _():
            f_ref[...] = t_ref[...] * n_nodes + s_ref[...]

        o_ref[0] = jnp.dot(x_ref[...], w_ref[0],
                           preferred_element_type=jnp.float32)

    return pl.pallas_call(
        body,
        grid=(R,),
        in_specs=[
            pl.BlockSpec((N, D), lambda r: (0, 0)),
            pl.BlockSpec((1, D, H), lambda r: (r, 0, 0)),
            pl.BlockSpec((EB, 128), lambda r: (0, 0)),
            pl.BlockSpec((EB, 128), lambda r: (0, 0)),
        ],
        out_specs=[
            pl.BlockSpec((1, N, H), lambda r: (r, 0, 0)),
            pl.BlockSpec((EB, 128), lambda r: (0, 0)),
        ],
        out_shape=[
            jax.ShapeDtypeStruct((R, N, H), jnp.float32),
            jax.ShapeDtypeStruct((EB, 128), jnp.int32),
        ],
    )(x, W, typ2d, src2d)


def _post_trans(acc, x, Wself, b, g, bb, W1, typ2d, src2d, n_nodes):
    R, D, H = W1.shape
    N = x.shape[0]
    EB = typ2d.shape[0]

    def body(acc_ref, x_ref, w_ref, b_ref, g_ref, bb_ref, w1_ref,
             t_ref, s_ref, o_ref, h_ref, f_ref, hs_ref):
        @pl.when(pl.program_id(0) == 0)
        def _():
            f_ref[...] = t_ref[...] * n_nodes + s_ref[...]
            h = (acc_ref[0, :N] + acc_ref[1, :N]
                 + jnp.dot(x_ref[...], w_ref[...],
                           preferred_element_type=jnp.float32)
                 + b_ref[...])
            mu = jnp.mean(h, axis=0, keepdims=True)
            var = jnp.mean((h - mu) ** 2, axis=0, keepdims=True)
            h = (h - mu) * lax.rsqrt(var + 1e-5) * g_ref[...] + bb_ref[...]
            h = jnp.where(h > 0, h, jnp.exp(jnp.minimum(h, 0.0)) - 1.0)
            hs_ref[...] = h
            h_ref[...] = h

        o_ref[0] = jnp.dot(hs_ref[...], w1_ref[0],
                           preferred_element_type=jnp.float32)

    return pl.pallas_call(
        body,
        grid=(R,),
        in_specs=[
            pl.BlockSpec(acc.shape, lambda r: (0, 0, 0)),
            pl.BlockSpec((N, D), lambda r: (0, 0)),
            pl.BlockSpec(Wself.shape, lambda r: (0, 0)),
            pl.BlockSpec((1, H), lambda r: (0, 0)),
            pl.BlockSpec((1, H), lambda r: (0, 0)),
            pl.BlockSpec((1, H), lambda r: (0, 0)),
            pl.BlockSpec((1, D, H), lambda r: (r, 0, 0)),
            pl.BlockSpec((EB, 128), lambda r: (0, 0)),
            pl.BlockSpec((EB, 128), lambda r: (0, 0)),
        ],
        out_specs=[
            pl.BlockSpec((1, N, H), lambda r: (r, 0, 0)),
            pl.BlockSpec((N, H), lambda r: (0, 0)),
            pl.BlockSpec((EB, 128), lambda r: (0, 0)),
        ],
        out_shape=[
            jax.ShapeDtypeStruct((R, N, H), jnp.float32),
            jax.ShapeDtypeStruct((N, H), jnp.float32),
            jax.ShapeDtypeStruct((EB, 128), jnp.int32),
        ],
        scratch_shapes=[pltpu.VMEM((N, H), jnp.float32)],
    )(acc, x, Wself, b, g, bb, W1, typ2d, src2d)


def _sc_degree(dst3d, zdeg, n_pad):
    _, K2, Cc = dst3d.shape
    KS = K2 // 4
    RPW = n_pad // NS

    mesh = plsc.VectorSubcoreMesh(core_axis_name="c", subcore_axis_name="s")

    @functools.partial(
        pl.kernel,
        mesh=mesh,
        out_type=jax.ShapeDtypeStruct((NC * n_pad,), jnp.float32),
        scratch_types=[
            pltpu.VMEM((KS, Cc), jnp.int32),
            pltpu.VMEM((Cc,), jnp.float32),
            pltpu.SemaphoreType.DMA,
            pltpu.VMEM_SHARED((n_pad,), jnp.float32),
        ],
    )
    def k(dst_hbm, zdeg_hbm, deg_out, dst_v, ones_v, sem_d, deg_sh):
        c = lax.axis_index("c")
        s = lax.axis_index("s")
        rbase = pl.multiple_of(s * RPW, 8)

        pltpu.sync_copy(zdeg_hbm.at[pl.ds(rbase, RPW)],
                        deg_sh.at[pl.ds(rbase, RPW)])

        @pl.loop(0, Cc, step=16)
        def _(i):
            ones_v[pl.ds(i, 16)] = jnp.ones((16,), jnp.float32)

        plsc.subcore_barrier()

        for t in range(4):
            base = pl.multiple_of(t * KS, 8)
            pltpu.sync_copy(dst_hbm.at[s, pl.ds(base, KS)], dst_v)

            @pl.loop(0, KS)
            def _(j):
                pltpu.async_copy(ones_v, deg_sh.at[dst_v.at[j]], sem_d,
                                 add=True)

            @pl.loop(0, KS)
            def _(j):
                pltpu.make_async_copy(ones_v, deg_sh.at[pl.ds(0, Cc)],
                                      sem_d).wait()

        plsc.subcore_barrier()

        out_base = pl.multiple_of(c * n_pad + s * RPW, 8)
        pltpu.sync_copy(deg_sh.at[pl.ds(rbase, RPW)],
                        deg_out.at[pl.ds(out_base, RPW)])

    return k(dst3d, zdeg)


def _sc_aggregate(trans_flat, flat3d, dst3d, deg1d, zacc, n_pad):
    RN, H = trans_flat.shape
    _, K2, Cc = flat3d.shape
    K = K2 // 2
    KS = K // 2
    RPW = n_pad // NS
    HG = H // 16

    mesh = plsc.VectorSubcoreMesh(core_axis_name="c", subcore_axis_name="s")

    @functools.partial(
        pl.kernel,
        mesh=mesh,
        out_type=jax.ShapeDtypeStruct((NC, n_pad, H), jnp.float32),
        scratch_types=[
            pltpu.VMEM((KS, Cc), jnp.int32),
            pltpu.VMEM((KS, Cc), jnp.int32),
            pltpu.VMEM((Cc, H), jnp.float32),
            pltpu.VMEM((Cc, H), jnp.float32),
            pltpu.VMEM((RPW,), jnp.float32),
            pltpu.SemaphoreType.DMA,
            pltpu.SemaphoreType.DMA,
            pltpu.SemaphoreType.DMA,
            pltpu.SemaphoreType.DMA,
            pltpu.VMEM_SHARED((n_pad, H), jnp.float32),
        ],
    )
    def k(trans_hbm, flat_hbm, dst_hbm, deg_hbm, zacc_hbm, acc_out,
          flat_v, dst_v, rows_a, rows_b, deg_v, sem_a, sem_b,
          sem_oa, sem_ob, acc_sh):
        c = lax.axis_index("c")
        s = lax.axis_index("s")
        rbase = pl.multiple_of(s * RPW, 8)

        pltpu.sync_copy(zacc_hbm.at[pl.ds(rbase, RPW)],
                        acc_sh.at[pl.ds(rbase, RPW)])

        pltpu.sync_copy(deg_hbm.at[pl.ds(pl.multiple_of(
            c * n_pad + s * RPW, 8), RPW)], deg_v)

        plsc.subcore_barrier()

        def gstart(j, rows_ref, sem):
            pltpu.async_copy(trans_hbm.at[flat_v.at[j]], rows_ref, sem)

        def gwait(rows_ref, sem):
            pltpu.make_async_copy(trans_hbm.at[flat_v.at[0]],
                                  rows_ref, sem).wait()

        for t in range(2):
            base = pl.multiple_of(c * K + t * KS, 8)
            pltpu.sync_copy(flat_hbm.at[s, pl.ds(base, KS)], flat_v)
            pltpu.sync_copy(dst_hbm.at[s, pl.ds(base, KS)], dst_v)
            gstart(0, rows_a, sem_a)

            @pl.loop(0, KS // 2)
            def _(p):
                j = p * 2
                gstart(j + 1, rows_b, sem_b)
                gwait(rows_a, sem_a)
                pltpu.sync_copy(rows_a, acc_sh.at[dst_v.at[j]], add=True)

                @pl.when(j + 2 < KS)
                def _():
                    gstart(j + 2, rows_a, sem_a)

                gwait(rows_b, sem_b)
                pltpu.sync_copy(rows_b, acc_sh.at[dst_v.at[j + 1]], add=True)

        plsc.subcore_barrier()

        NB = RPW // NBLK
        bufs = [rows_a, rows_b]
        isems = [sem_a, sem_b]
        osems = [sem_oa, sem_ob]

        def blk_off(b):
            return pl.multiple_of(rbase + b * NBLK, 8)

        def norm_block(buf, b):
            @pl.loop(0, NBLK // 16)
            def _(g):
                d = deg_v[pl.ds(b * NBLK + g * 16, 16)]
                rec = 1.0 / jnp.maximum(d, 1.0)
                dnums = lax.GatherDimensionNumbers(
                    offset_dims=(), collapsed_slice_dims=(0,),
                    start_index_map=(0,))
                for l in range(16):
                    r = g * 16 + l
                    rl = lax.gather(
                        rec, jnp.full((16, 1), l, jnp.int32), dnums,
                        slice_sizes=(1,),
                        mode=lax.GatherScatterMode.PROMISE_IN_BOUNDS)
                    for hgrp in range(HG):
                        buf[r, pl.ds(hgrp * 16, 16)] = (
                            buf[r, pl.ds(hgrp * 16, 16)] * rl)

        pltpu.async_copy(acc_sh.at[pl.ds(blk_off(0), NBLK)], rows_a, sem_a)
        for b in range(NB):
            buf, si, so = bufs[b % 2], isems[b % 2], osems[b % 2]
            pltpu.make_async_copy(acc_sh.at[pl.ds(0, NBLK)], buf, si).wait()
            if b + 1 < NB:
                nbuf = bufs[(b + 1) % 2]
                if b >= 1:
                    pltpu.make_async_copy(
                        nbuf, acc_out.at[c, pl.ds(rbase, NBLK)],
                        osems[(b + 1) % 2]).wait()
                pltpu.async_copy(acc_sh.at[pl.ds(blk_off(b + 1), NBLK)],
                                 nbuf, isems[(b + 1) % 2])
            norm_block(buf, b)
            pltpu.async_copy(buf, acc_out.at[c, pl.ds(blk_off(b), NBLK)], so)

        pltpu.make_async_copy(bufs[(NB - 2) % 2],
                              acc_out.at[c, pl.ds(rbase, NBLK)],
                              osems[(NB - 2) % 2]).wait()
        pltpu.make_async_copy(bufs[(NB - 1) % 2],
                              acc_out.at[c, pl.ds(rbase, NBLK)],
                              osems[(NB - 1) % 2]).wait()

    return k(trans_flat, flat3d, dst3d, deg1d, zacc)


def _final(acc, x, Wself, b, g, bb, mW1, mb1, mg, mbb, mW2, mb2):
    N = x.shape[0]
    D_OUT = mW2.shape[1]

    def body(acc_ref, x_ref, w_ref, b_ref, g_ref, bb_ref,
             mW1_ref, mb1_ref, mg_ref, mbb_ref, mW2_ref, mb2_ref, o_ref):
        h = (acc_ref[0, :N] + acc_ref[1, :N]
             + jnp.dot(x_ref[...], w_ref[...],
                       preferred_element_type=jnp.float32)
             + b_ref[...])
        mu = jnp.mean(h, axis=0, keepdims=True)
        var = jnp.mean((h - mu) ** 2, axis=0, keepdims=True)
        h = (h - mu) * lax.rsqrt(var + 1e-5) * g_ref[...] + bb_ref[...]
        h = jnp.where(h > 0, h, jnp.exp(jnp.minimum(h, 0.0)) - 1.0)
        m = jnp.dot(h, mW1_ref[...],
                    preferred_element_type=jnp.float32) + mb1_ref[...]
        mu2 = jnp.mean(m, axis=0, keepdims=True)
        var2 = jnp.mean((m - mu2) ** 2, axis=0, keepdims=True)
        m = (m - mu2) * lax.rsqrt(var2 + 1e-5) * mg_ref[...] + mbb_ref[...]
        m = jnp.maximum(m, 0.0)
        o_ref[...] = jnp.dot(m, mW2_ref[...],
                             preferred_element_type=jnp.float32) + mb2_ref[...]

    return pl.pallas_call(
        body,
        out_shape=jax.ShapeDtypeStruct((N, D_OUT), jnp.float32),
    )(acc, x, Wself, b, g, bb, mW1, mb1, mg, mbb, mW2, mb2)


def kernel(x, edge_index0, edge_type0, edge_index1, edge_type1,
           W0, Wself0, b0, bn0_g, bn0_b,
           W1, Wself1, b1, bn1_g, bn1_b,
           mlp_W1, mlp_b1, mlp_bn_g, mlp_bn_b, mlp_W2, mlp_b2):
    N = x.shape[0]
    E = edge_type0.shape[0]
    R, _, H = W0.shape
    NPAD = ((N + 16 * NS - 1) // (16 * NS)) * (16 * NS)
    K = (E + NS * 2 * C - 1) // (NS * 2 * C)
    K = ((K + 7) // 8) * 8
    EPAD = NS * 2 * K * C

    zacc = jnp.zeros((NPAD, H), jnp.float32)
    zdeg = jnp.zeros((NPAD,), jnp.float32)

    def edges_prep(edge_index, edge_type):
        pad = EPAD - E
        pad_iota = lax.iota(jnp.int32, pad)
        src = jnp.concatenate([edge_index[0], pad_iota % N])
        typ = jnp.concatenate([edge_type, jnp.zeros((pad,), jnp.int32)])
        dst = jnp.concatenate([edge_index[1], N + pad_iota % (NPAD - N)])
        return (typ.reshape(EPAD // C, C), src.reshape(EPAD // C, C),
                dst.reshape(NS, 2 * K, C))

    typ0_2d, src0_2d, dst0 = edges_prep(edge_index0, edge_type0)
    typ1_2d, src1_2d, dst1 = edges_prep(edge_index1, edge_type1)

    r1h = lambda v: v.reshape(1, -1)

    deg0 = _sc_degree(dst0, zdeg, NPAD)
    trans0, flat0 = _trans(x, W0, typ0_2d, src0_2d, N)
    acc0 = _sc_aggregate(trans0.reshape(R * N, H),
                         flat0.reshape(NS, 2 * K, C), dst0, deg0, zacc, NPAD)
    deg1 = _sc_degree(dst1, zdeg, NPAD)
    trans1, h, flat1 = _post_trans(acc0, x, Wself0, r1h(b0), r1h(bn0_g),
                                   r1h(bn0_b), W1, typ1_2d, src1_2d, N)
    acc1 = _sc_aggregate(trans1.reshape(R * N, H),
                         flat1.reshape(NS, 2 * K, C), dst1, deg1, zacc, NPAD)
    out = _final(acc1, h, Wself1, r1h(b1), r1h(bn1_g), r1h(bn1_b),
                 mlp_W1, r1h(mlp_b1), r1h(mlp_bn_g), r1h(mlp_bn_b),
                 mlp_W2, r1h(mlp_b2))
    return out
